# Initial kernel scaffold; baseline (speedup 1.0000x reference)
import jax
import jax.numpy as jnp
from jax.experimental import pallas as pl


def kernel(idsTensor, emb_table, W1, b1, gamma, beta, W2, b2):
    raise NotImplementedError("write your pallas kernel here")


# speedup vs baseline: 2.4829x; 2.4829x over previous
import functools

import jax
import jax.numpy as jnp
from jax import lax
from jax.experimental import pallas as pl
from jax.experimental.pallas import tpu as pltpu
from jax.experimental.pallas import tpu_sc as plsc


def _make_sc_gather(num_rows_total, emb, n_workers, n_chunks, chunk):
    mesh = plsc.VectorSubcoreMesh(core_axis_name="c", subcore_axis_name="s")
    rows_per_w = n_chunks * chunk
    assert rows_per_w * n_workers == num_rows_total

    @functools.partial(
        pl.kernel,
        mesh=mesh,
        out_type=jax.ShapeDtypeStruct((num_rows_total, emb), jnp.float32),
        scratch_types=[
            pltpu.VMEM((n_chunks, chunk), jnp.int32),
            pltpu.VMEM((chunk, emb), jnp.float32),
            pltpu.VMEM((chunk, emb), jnp.float32),
            pltpu.SemaphoreType.DMA,
            pltpu.SemaphoreType.DMA,
        ],
    )
    def sc_gather(table_hbm, idx_hbm, out_hbm, idx_v, rows0, rows1, sem0, sem1):
        wid = lax.axis_index("s") * 2 + lax.axis_index("c")
        base = wid * rows_per_w
        pltpu.sync_copy(idx_hbm.at[wid], idx_v)
        rows = (rows0, rows1)
        sems = (sem0, sem1)
        handles = [None, None]
        for j in range(min(2, n_chunks)):
            handles[j] = pltpu.async_copy(
                table_hbm.at[idx_v.at[j]], rows[j], sems[j])
        for j in range(n_chunks):
            s = j % 2
            handles[s].wait()
            pltpu.sync_copy(rows[s], out_hbm.at[pl.ds(base + j * chunk, chunk)])
            if j + 2 < n_chunks:
                handles[s] = pltpu.async_copy(
                    table_hbm.at[idx_v.at[j + 2]], rows[s], sems[s])

    return sc_gather


def _mlp_body(n_blocks, bb, f1_ref, f2_ref, sd_ref, w1a_ref, w1b_ref, w1c_ref,
              b1_ref, gamma_ref, beta_ref, w2_ref, b2_ref, out_ref, x_scratch):
    i = pl.program_id(0)
    xb = (
        jnp.dot(f1_ref[:], w1a_ref[:], preferred_element_type=jnp.float32)
        + jnp.dot(f2_ref[:], w1b_ref[:], preferred_element_type=jnp.float32)
        + sd_ref[:] * w1c_ref[:]
        + b1_ref[:]
    )
    x_scratch[pl.ds(i * bb, bb), :] = xb

    @pl.when(i == n_blocks - 1)
    def _finish():
        x = x_scratch[:]
        mean = jnp.mean(x, axis=0, keepdims=True)
        var = jnp.mean((x - mean) ** 2, axis=0, keepdims=True)
        xn = (x - mean) * lax.rsqrt(var + 1e-5) * gamma_ref[:] + beta_ref[:]
        xn = jnp.where(xn >= 0, xn, 0.1 * xn)
        o = jnp.sum(xn * w2_ref[:], axis=1, keepdims=True) + b2_ref[:]
        out_ref[:] = jax.nn.sigmoid(o)


def kernel(idsTensor, emb_table, W1, b1, gamma, beta, W2, b2):
    bsz, _ = idsTensor.shape
    n_teams, emb = emb_table.shape
    hid = W1.shape[0]

    n_workers = 32
    chunk = 128
    total_rows = 2 * bsz
    n_chunks = total_rows // (n_workers * chunk)

    idx = jnp.concatenate(
        [idsTensor[:, 0], idsTensor[:, 1]]).astype(jnp.int32)
    idx = idx.reshape(n_workers, n_chunks, chunk)

    sc_gather = _make_sc_gather(total_rows, emb, n_workers, n_chunks, chunk)
    feats = sc_gather(emb_table, idx)

    sd = idsTensor[:, 2:3]
    w1aT = W1[:, :emb].T
    w1bT = W1[:, emb:2 * emb].T
    w1c = W1[:, 2 * emb:].T
    b1r = b1.reshape(1, hid)
    gammar = gamma.reshape(1, hid)
    betar = beta.reshape(1, hid)
    w2r = W2.reshape(1, hid)
    b2r = b2.reshape(1, 1)

    bb = 2048
    n_blocks = bsz // bb

    full = lambda shape: pl.BlockSpec(shape, lambda i: (0, 0))
    out = pl.pallas_call(
        functools.partial(_mlp_body, n_blocks, bb),
        grid=(n_blocks,),
        in_specs=[
            pl.BlockSpec((bb, emb), lambda i: (i, 0)),
            pl.BlockSpec((bb, emb), lambda i: (i + n_blocks, 0)),
            pl.BlockSpec((bb, 1), lambda i: (i, 0)),
            full((emb, hid)),
            full((emb, hid)),
            full((1, hid)),
            full((1, hid)),
            full((1, hid)),
            full((1, hid)),
            full((1, hid)),
            full((1, 1)),
        ],
        out_specs=pl.BlockSpec((bsz, 1), lambda i: (0, 0)),
        out_shape=jax.ShapeDtypeStruct((bsz, 1), jnp.float32),
        scratch_shapes=[pltpu.VMEM((bsz, hid), jnp.float32)],
        compiler_params=pltpu.CompilerParams(
            dimension_semantics=("arbitrary",)),
    )(feats, feats, sd, w1aT, w1bT, w1c, b1r, gammar, betar, w2r, b2r)
    return out


# speedup vs baseline: 2.5057x; 1.0092x over previous
import functools

import jax
import jax.numpy as jnp
from jax import lax
from jax.experimental import pallas as pl
from jax.experimental.pallas import tpu as pltpu
from jax.experimental.pallas import tpu_sc as plsc


def _make_sc_gather(num_rows_total, emb, n_workers, n_chunks, chunk):
    mesh = plsc.VectorSubcoreMesh(core_axis_name="c", subcore_axis_name="s")
    rows_per_w = n_chunks * chunk
    assert rows_per_w * n_workers == num_rows_total

    nbuf = 4

    @functools.partial(
        pl.kernel,
        mesh=mesh,
        out_type=jax.ShapeDtypeStruct((num_rows_total, emb), jnp.float32),
        scratch_types=[
            pltpu.VMEM((n_chunks, chunk), jnp.int32),
            [pltpu.VMEM((chunk, emb), jnp.float32) for _ in range(nbuf)],
            [pltpu.SemaphoreType.DMA for _ in range(nbuf)],
            [pltpu.SemaphoreType.DMA for _ in range(nbuf)],
        ],
    )
    def sc_gather(table_hbm, idx_hbm, out_hbm, idx_v, rows, gsems, ssems):
        wid = lax.axis_index("s") * 2 + lax.axis_index("c")
        base = wid * rows_per_w
        pltpu.sync_copy(idx_hbm.at[wid], idx_v)
        gh = [None] * nbuf
        sh = [None] * nbuf
        for j in range(min(nbuf, n_chunks)):
            gh[j] = pltpu.async_copy(
                table_hbm.at[idx_v.at[j]], rows[j], gsems[j])
        for j in range(n_chunks):
            s = j % nbuf
            gh[s].wait()
            sh[s] = pltpu.async_copy(
                rows[s], out_hbm.at[pl.ds(base + j * chunk, chunk)], ssems[s])
            nxt = j - 1 + nbuf
            if j >= 1 and nxt < n_chunks:
                t = nxt % nbuf
                sh[t].wait()
                gh[t] = pltpu.async_copy(
                    table_hbm.at[idx_v.at[nxt]], rows[t], gsems[t])
        for j in range(max(0, n_chunks - nbuf), n_chunks):
            sh[j % nbuf].wait()

    return sc_gather


def _mlp_body(n_blocks, bb, f1_ref, f2_ref, sd_ref, w1a_ref, w1b_ref, w1c_ref,
              b1_ref, gamma_ref, beta_ref, w2_ref, b2_ref, out_ref, x_scratch):
    i = pl.program_id(0)
    xb = (
        jnp.dot(f1_ref[:], w1a_ref[:], preferred_element_type=jnp.float32)
        + jnp.dot(f2_ref[:], w1b_ref[:], preferred_element_type=jnp.float32)
        + sd_ref[:] * w1c_ref[:]
        + b1_ref[:]
    )
    x_scratch[pl.ds(i * bb, bb), :] = xb

    @pl.when(i == n_blocks - 1)
    def _finish():
        x = x_scratch[:]
        mean = jnp.mean(x, axis=0, keepdims=True)
        var = jnp.mean((x - mean) ** 2, axis=0, keepdims=True)
        xn = (x - mean) * lax.rsqrt(var + 1e-5) * gamma_ref[:] + beta_ref[:]
        xn = jnp.where(xn >= 0, xn, 0.1 * xn)
        o = jnp.sum(xn * w2_ref[:], axis=1, keepdims=True) + b2_ref[:]
        out_ref[:] = jax.nn.sigmoid(o)


def kernel(idsTensor, emb_table, W1, b1, gamma, beta, W2, b2):
    bsz, _ = idsTensor.shape
    n_teams, emb = emb_table.shape
    hid = W1.shape[0]

    n_workers = 32
    chunk = 128
    total_rows = 2 * bsz
    n_chunks = total_rows // (n_workers * chunk)

    idx = jnp.concatenate(
        [idsTensor[:, 0], idsTensor[:, 1]]).astype(jnp.int32)
    idx = idx.reshape(n_workers, n_chunks, chunk)

    sc_gather = _make_sc_gather(total_rows, emb, n_workers, n_chunks, chunk)
    feats = sc_gather(emb_table, idx)

    sd = idsTensor[:, 2:3]
    w1aT = W1[:, :emb].T
    w1bT = W1[:, emb:2 * emb].T
    w1c = W1[:, 2 * emb:].T
    b1r = b1.reshape(1, hid)
    gammar = gamma.reshape(1, hid)
    betar = beta.reshape(1, hid)
    w2r = W2.reshape(1, hid)
    b2r = b2.reshape(1, 1)

    bb = 2048
    n_blocks = bsz // bb

    full = lambda shape: pl.BlockSpec(shape, lambda i: (0, 0))
    out = pl.pallas_call(
        functools.partial(_mlp_body, n_blocks, bb),
        grid=(n_blocks,),
        in_specs=[
            pl.BlockSpec((bb, emb), lambda i: (i, 0)),
            pl.BlockSpec((bb, emb), lambda i: (i + n_blocks, 0)),
            pl.BlockSpec((bb, 1), lambda i: (i, 0)),
            full((emb, hid)),
            full((emb, hid)),
            full((1, hid)),
            full((1, hid)),
            full((1, hid)),
            full((1, hid)),
            full((1, hid)),
            full((1, 1)),
        ],
        out_specs=pl.BlockSpec((bsz, 1), lambda i: (0, 0)),
        out_shape=jax.ShapeDtypeStruct((bsz, 1), jnp.float32),
        scratch_shapes=[pltpu.VMEM((bsz, hid), jnp.float32)],
        compiler_params=pltpu.CompilerParams(
            dimension_semantics=("arbitrary",)),
    )(feats, feats, sd, w1aT, w1bT, w1c, b1r, gammar, betar, w2r, b2r)
    return out


# speedup vs baseline: 3.5330x; 1.4100x over previous
import functools

import jax
import jax.numpy as jnp
from jax import lax
from jax.experimental import pallas as pl
from jax.experimental.pallas import tpu as pltpu
from jax.experimental.pallas import tpu_sc as plsc


def _make_sc_gather(bsz, emb, n_workers, n_chunks, chunk):
    mesh = plsc.VectorSubcoreMesh(core_axis_name="c", subcore_axis_name="s")
    rows_per_w = n_chunks * chunk
    half = n_workers // 2
    assert rows_per_w * half == bsz

    nbuf = 4

    @functools.partial(
        pl.kernel,
        mesh=mesh,
        out_type=jax.ShapeDtypeStruct((bsz, 2 * emb), jnp.float32),
        scratch_types=[
            pltpu.VMEM((n_chunks, chunk), jnp.int32),
            [pltpu.VMEM((chunk, emb), jnp.float32) for _ in range(nbuf)],
            [pltpu.SemaphoreType.DMA for _ in range(nbuf)],
            [pltpu.SemaphoreType.DMA for _ in range(nbuf)],
        ],
    )
    def sc_gather(table_hbm, idx_hbm, out_hbm, idx_v, rows, gsems, ssems):
        wid = lax.axis_index("s") * 2 + lax.axis_index("c")
        base = (wid % half) * rows_per_w
        col = (wid // half) * emb
        pltpu.sync_copy(idx_hbm.at[wid], idx_v)
        gh = [None] * nbuf
        sh = [None] * nbuf

        def store_dst(j):
            return out_hbm.at[pl.ds(base + j * chunk, chunk), pl.ds(col, emb)]

        for j in range(min(nbuf, n_chunks)):
            gh[j] = pltpu.async_copy(
                table_hbm.at[idx_v.at[j]], rows[j], gsems[j])
        for j in range(n_chunks):
            s = j % nbuf
            gh[s].wait()
            sh[s] = pltpu.async_copy(rows[s], store_dst(j), ssems[s])
            nxt = j - 1 + nbuf
            if j >= 1 and nxt < n_chunks:
                t = nxt % nbuf
                sh[t].wait()
                gh[t] = pltpu.async_copy(
                    table_hbm.at[idx_v.at[nxt]], rows[t], gsems[t])
        for j in range(max(0, n_chunks - nbuf), n_chunks):
            sh[j % nbuf].wait()

    return sc_gather


def _mlp_body(n_blocks, bb, feats_ref, sd_ref, w1ab_ref, w1c_ref, b1_ref,
              gamma_ref, beta_ref, w2_ref, b2_ref, out_ref, x_scratch):
    i = pl.program_id(0)
    xbt = lax.dot_general(
        w1ab_ref[:], feats_ref[:], (((1,), (1,)), ((), ())),
        preferred_element_type=jnp.float32,
    )
    x_scratch[:, pl.ds(i * bb, bb)] = (
        xbt + sd_ref[:] * w1c_ref[:] + b1_ref[:])

    @pl.when(i == n_blocks - 1)
    def _finish():
        x = x_scratch[:]
        mean = jnp.mean(x, axis=1, keepdims=True)
        var = jnp.mean((x - mean) ** 2, axis=1, keepdims=True)
        xn = (x - mean) * lax.rsqrt(var + 1e-5) * gamma_ref[:] + beta_ref[:]
        xn = jnp.where(xn >= 0, xn, 0.1 * xn)
        o = jnp.sum(xn * w2_ref[:], axis=0, keepdims=True) + b2_ref[:]
        out_ref[:] = jax.nn.sigmoid(o)


def kernel(idsTensor, emb_table, W1, b1, gamma, beta, W2, b2):
    bsz, _ = idsTensor.shape
    n_teams, emb = emb_table.shape
    hid = W1.shape[0]

    n_workers = 32
    chunk = 128
    n_chunks = 2 * bsz // (n_workers * chunk)

    idx = jnp.concatenate(
        [idsTensor[:, 0], idsTensor[:, 1]]).astype(jnp.int32)
    idx = idx.reshape(n_workers, n_chunks, chunk)

    sc_gather = _make_sc_gather(bsz, emb, n_workers, n_chunks, chunk)
    feats = sc_gather(emb_table, idx)

    sd = idsTensor[:, 2].reshape(1, bsz)
    w1ab = W1[:, :2 * emb]
    w1c = W1[:, 2 * emb:]
    b1c = b1.reshape(hid, 1)
    gammac = gamma.reshape(hid, 1)
    betac = beta.reshape(hid, 1)
    w2c = W2.reshape(hid, 1)
    b2r = b2.reshape(1, 1)

    bb = 2048
    n_blocks = bsz // bb

    full = lambda shape: pl.BlockSpec(shape, lambda i: (0, 0))
    out = pl.pallas_call(
        functools.partial(_mlp_body, n_blocks, bb),
        grid=(n_blocks,),
        in_specs=[
            pl.BlockSpec((bb, 2 * emb), lambda i: (i, 0)),
            pl.BlockSpec((1, bb), lambda i: (0, i)),
            full((hid, 2 * emb)),
            full((hid, 1)),
            full((hid, 1)),
            full((hid, 1)),
            full((hid, 1)),
            full((hid, 1)),
            full((1, 1)),
        ],
        out_specs=pl.BlockSpec((1, bsz), lambda i: (0, 0)),
        out_shape=jax.ShapeDtypeStruct((1, bsz), jnp.float32),
        scratch_shapes=[pltpu.VMEM((hid, bsz), jnp.float32)],
        compiler_params=pltpu.CompilerParams(
            dimension_semantics=("arbitrary",)),
    )(feats, sd, w1ab, w1c, b1c, gammac, betac, w2c, b2r)
    return out.reshape(bsz, 1)


# speedup vs baseline: 3.8082x; 1.0779x over previous
import functools

import jax
import jax.numpy as jnp
from jax import lax
from jax.experimental import pallas as pl
from jax.experimental.pallas import tpu as pltpu
from jax.experimental.pallas import tpu_sc as plsc


def _make_sc_gather(bsz, emb, n_workers, n_chunks, chunk):
    mesh = plsc.VectorSubcoreMesh(core_axis_name="c", subcore_axis_name="s")
    rows_per_w = n_chunks * chunk
    half = n_workers // 2
    assert rows_per_w * half == bsz

    nbuf = 6

    @functools.partial(
        pl.kernel,
        mesh=mesh,
        out_type=jax.ShapeDtypeStruct((bsz, 2 * emb), jnp.float32),
        scratch_types=[
            pltpu.VMEM((n_chunks, chunk), jnp.int32),
            [pltpu.VMEM((chunk, emb), jnp.float32) for _ in range(nbuf)],
            [pltpu.SemaphoreType.DMA for _ in range(nbuf)],
            [pltpu.SemaphoreType.DMA for _ in range(nbuf)],
        ],
    )
    def sc_gather(table_hbm, idx_hbm, out_hbm, idx_v, rows, gsems, ssems):
        wid = lax.axis_index("s") * 2 + lax.axis_index("c")
        base = (wid % half) * rows_per_w
        col = (wid // half) * emb
        pltpu.sync_copy(idx_hbm.at[wid], idx_v)
        gh = [None] * nbuf
        sh = [None] * nbuf

        def store_dst(j):
            return out_hbm.at[pl.ds(base + j * chunk, chunk), pl.ds(col, emb)]

        for j in range(min(nbuf, n_chunks)):
            gh[j] = pltpu.async_copy(
                table_hbm.at[idx_v.at[j]], rows[j], gsems[j])
        for j in range(n_chunks):
            s = j % nbuf
            gh[s].wait()
            sh[s] = pltpu.async_copy(rows[s], store_dst(j), ssems[s])
            nxt = j - 1 + nbuf
            if j >= 1 and nxt < n_chunks:
                t = nxt % nbuf
                sh[t].wait()
                gh[t] = pltpu.async_copy(
                    table_hbm.at[idx_v.at[nxt]], rows[t], gsems[t])
        for j in range(max(0, n_chunks - nbuf), n_chunks):
            sh[j % nbuf].wait()

    return sc_gather


def _mlp_body(n_blocks, bb, feats_ref, sd_ref, w1ab_ref, w1c_ref, b1_ref,
              gamma_ref, beta_ref, w2_ref, b2_ref, out_ref, x_scratch):
    i = pl.program_id(0)
    xbt = lax.dot_general(
        w1ab_ref[:], feats_ref[:], (((1,), (1,)), ((), ())),
        preferred_element_type=jnp.float32,
    )
    x_scratch[:, pl.ds(i * bb, bb)] = (
        xbt + sd_ref[:] * w1c_ref[:] + b1_ref[:])

    @pl.when(i == n_blocks - 1)
    def _finish():
        x = x_scratch[:]
        mean = jnp.mean(x, axis=1, keepdims=True)
        var = jnp.mean((x - mean) ** 2, axis=1, keepdims=True)
        xn = (x - mean) * lax.rsqrt(var + 1e-5) * gamma_ref[:] + beta_ref[:]
        xn = jnp.where(xn >= 0, xn, 0.1 * xn)
        o = jnp.sum(xn * w2_ref[:], axis=0, keepdims=True) + b2_ref[:]
        out_ref[:] = jax.nn.sigmoid(o)


def kernel(idsTensor, emb_table, W1, b1, gamma, beta, W2, b2):
    bsz, _ = idsTensor.shape
    n_teams, emb = emb_table.shape
    hid = W1.shape[0]

    n_workers = 32
    chunk = 128
    n_chunks = 2 * bsz // (n_workers * chunk)

    idx = jnp.concatenate(
        [idsTensor[:, 0], idsTensor[:, 1]]).astype(jnp.int32)
    idx = idx.reshape(n_workers, n_chunks, chunk)

    sc_gather = _make_sc_gather(bsz, emb, n_workers, n_chunks, chunk)
    feats = sc_gather(emb_table, idx)

    sd = idsTensor[:, 2].reshape(1, bsz)
    w1ab = W1[:, :2 * emb]
    w1c = W1[:, 2 * emb:]
    b1c = b1.reshape(hid, 1)
    gammac = gamma.reshape(hid, 1)
    betac = beta.reshape(hid, 1)
    w2c = W2.reshape(hid, 1)
    b2r = b2.reshape(1, 1)

    bb = 4096
    n_blocks = bsz // bb

    full = lambda shape: pl.BlockSpec(shape, lambda i: (0, 0))
    out = pl.pallas_call(
        functools.partial(_mlp_body, n_blocks, bb),
        grid=(n_blocks,),
        in_specs=[
            pl.BlockSpec((bb, 2 * emb), lambda i: (i, 0)),
            pl.BlockSpec((1, bb), lambda i: (0, i)),
            full((hid, 2 * emb)),
            full((hid, 1)),
            full((hid, 1)),
            full((hid, 1)),
            full((hid, 1)),
            full((hid, 1)),
            full((1, 1)),
        ],
        out_specs=pl.BlockSpec((1, bsz), lambda i: (0, 0)),
        out_shape=jax.ShapeDtypeStruct((1, bsz), jnp.float32),
        scratch_shapes=[pltpu.VMEM((hid, bsz), jnp.float32)],
        compiler_params=pltpu.CompilerParams(
            dimension_semantics=("arbitrary",)),
    )(feats, sd, w1ab, w1c, b1c, gammac, betac, w2c, b2r)
    return out.reshape(bsz, 1)
